# Initial kernel scaffold; baseline (speedup 1.0000x reference)
import jax
import jax.numpy as jnp
from jax.experimental import pallas as pl


def kernel(k_cache, v_cache, pos_ids, k, v):
    raise NotImplementedError("write your pallas kernel here")


# speedup vs baseline: 2.1355x; 2.1355x over previous
import jax
import jax.numpy as jnp
from jax.experimental import pallas as pl
from jax.experimental.pallas import tpu as pltpu

_N_HEADS = 32
_MAX_CTX = 8192
_HDIM = 128
_QLEN = 16
_BLK = 2048
_NBLK = _MAX_CTX // _BLK


def _merge_kernel(pos_ref, kc_ref, vc_ref, k_ref, v_ref, ko_ref, vo_ref):
    ko_ref[...] = kc_ref[...]
    vo_ref[...] = vc_ref[...]
    base = pl.program_id(1) * _BLK
    for i in range(_QLEN):
        p = pos_ref[i]

        @pl.when(jnp.logical_and(p >= base, p < base + _BLK))
        def _():
            ko_ref[0, 0, p - base, :] = k_ref[0, 0, i, :]
            vo_ref[0, 0, p - base, :] = v_ref[0, 0, i, :]


def kernel(k_cache, v_cache, pos_ids, k, v):
    cache_spec = pl.BlockSpec((1, 1, _BLK, _HDIM), lambda h, j: (0, h, j, 0))
    kv_spec = pl.BlockSpec((1, 1, _QLEN, _HDIM), lambda h, j: (0, h, 0, 0))
    pos_spec = pl.BlockSpec(memory_space=pltpu.SMEM)
    ko, vo = pl.pallas_call(
        _merge_kernel,
        grid=(_N_HEADS, _NBLK),
        in_specs=[pos_spec, cache_spec, cache_spec, kv_spec, kv_spec],
        out_specs=[cache_spec, cache_spec],
        out_shape=[
            jax.ShapeDtypeStruct(k_cache.shape, k_cache.dtype),
            jax.ShapeDtypeStruct(v_cache.shape, v_cache.dtype),
        ],
        compiler_params=pltpu.CompilerParams(
            dimension_semantics=("parallel", "parallel"),
        ),
    )(pos_ids.astype(jnp.int32), k_cache, v_cache, k, v)
    return (ko, vo)
